# baseline (speedup 1.0000x reference)
import functools

import jax
import jax.numpy as jnp
from jax.experimental import pallas as pl
from jax.experimental.pallas import tpu as pltpu


def _mm_bf16_kernel(a_ref, b_ref, o_ref):
    o_ref[...] = jnp.dot(
        a_ref[...].astype(jnp.bfloat16), b_ref[...].astype(jnp.bfloat16),
        preferred_element_type=jnp.float32).astype(o_ref.dtype)


def _p1_kernel(s1_ref, adj_ref, h_ref):
    h_ref[...] = jnp.maximum(
        jnp.dot(adj_ref[...].astype(jnp.bfloat16), s1_ref[...],
                preferred_element_type=jnp.float32),
        0.0).astype(jnp.bfloat16)


def _p2_kernel(b_ref, adj_ref, o_ref, *, z):
    l = jnp.dot(adj_ref[...].astype(jnp.bfloat16), b_ref[...],
                preferred_element_type=jnp.float32)
    o_ref[0, 0, 0] = jnp.sum(l[:, z:])
    o_ref[0, 0, 1] = jnp.sum(jnp.exp(l[:, :z]) - 1.0)


def _pick_bm(n):
    for bm in (200, 80, 40, 16, 8):
        if n % bm == 0:
            return bm
    return n


def _small_mm(a, b, out_dtype):
    return pl.pallas_call(
        _mm_bf16_kernel,
        out_shape=jax.ShapeDtypeStruct((a.shape[0], b.shape[1]), out_dtype),
    )(a, b)


@jax.jit
def kernel(x, adj, W1, W2, W3):
    n, d = x.shape
    h_dim = W1.shape[1]
    z = W2.shape[1]
    bm = _pick_bm(n)
    nb = n // bm
    parallel = pltpu.CompilerParams(dimension_semantics=("parallel",))

    s1 = _small_mm(x, W1, jnp.bfloat16)

    hidden1 = pl.pallas_call(
        _p1_kernel,
        grid=(nb,),
        in_specs=[
            pl.BlockSpec((n, h_dim), lambda i: (0, 0)),
            pl.BlockSpec((bm, n), lambda i: (i, 0)),
        ],
        out_specs=pl.BlockSpec((bm, h_dim), lambda i: (i, 0)),
        out_shape=jax.ShapeDtypeStruct((n, h_dim), jnp.bfloat16),
        compiler_params=parallel,
    )(s1, adj)

    w23 = jnp.concatenate([W3, W2], axis=1)
    b = _small_mm(hidden1, w23, jnp.bfloat16)

    partials = pl.pallas_call(
        functools.partial(_p2_kernel, z=z),
        grid=(nb,),
        in_specs=[
            pl.BlockSpec((n, 2 * z), lambda i: (0, 0)),
            pl.BlockSpec((bm, n), lambda i: (i, 0)),
        ],
        out_specs=pl.BlockSpec((1, 1, 2), lambda i: (i, 0, 0),
                               memory_space=pltpu.SMEM),
        out_shape=jax.ShapeDtypeStruct((nb, 1, 2), jnp.float32),
        compiler_params=parallel,
    )(b, adj)

    nz = float(n * z)
    z_mu = jnp.sum(partials[:, 0, 0]) / nz
    z_logvar = jnp.log(1.0 + jnp.sum(partials[:, 0, 1]) / nz)
    return (z_mu, z_logvar)
